# baseline (speedup 1.0000x reference)
import functools

import jax
import jax.numpy as jnp
from jax import lax
from jax.experimental import pallas as pl
from jax.experimental.pallas import tpu as pltpu
from jax.experimental.pallas import tpu_sc as plsc

_NC = 2
_NS = 16
_NW = _NC * _NS
_L = 16

_CH = 512
_IDX_PER_DMA = 128


def _make_kernel(B, V, NP, D):
    per_w = B // _NW
    n_chunks = per_w // _CH
    n_dma = _CH // _IDX_PER_DMA
    mesh = plsc.VectorSubcoreMesh(core_axis_name="c", subcore_axis_name="s")

    @functools.partial(
        pl.kernel,
        out_type=jax.ShapeDtypeStruct((B, D), jnp.float32),
        mesh=mesh,
        compiler_params=pltpu.CompilerParams(
            needs_layout_passes=False, use_tc_tiling_on_sc=False
        ),
        scratch_types=[
            pltpu.VMEM((_CH,), jnp.int32),
            pltpu.VMEM((_CH,), jnp.int32),
            pltpu.VMEM((_CH, D), jnp.float32),
            pltpu.VMEM((NP, D), jnp.float32),
            pltpu.SemaphoreType.DMA,
        ],
    )
    def k(idx_hbm, embed_hbm, ne_hbm, out_hbm, idx_v, safe_v, rows_v, ne_v, sem):
        wid = lax.axis_index("s") * _NC + lax.axis_index("c")
        base = wid * per_w
        pltpu.sync_copy(ne_hbm, ne_v)

        def chunk_body(g, _):
            cbase = base + g * _CH
            pltpu.sync_copy(idx_hbm.at[pl.ds(cbase, _CH)], idx_v)

            def remap(i, _):
                v = idx_v[pl.ds(i * _L, _L)]
                safe_v[pl.ds(i * _L, _L)] = jnp.where(v >= V, 0, v)
                return 0

            lax.fori_loop(0, _CH // _L, remap, 0, unroll=4)

            copies = []
            for j in range(n_dma):
                copies.append(
                    pltpu.async_copy(
                        embed_hbm.at[safe_v.at[pl.ds(j * _IDX_PER_DMA, _IDX_PER_DMA)]],
                        rows_v.at[pl.ds(j * _IDX_PER_DMA, _IDX_PER_DMA)],
                        sem,
                    )
                )
            for c in copies:
                c.wait()

            def fixup(i, _):
                v = idx_v[pl.ds(i * _L, _L)]
                m = v >= V
                p = jnp.where(m, v - V, 0)

                @pl.when(jnp.max(v) >= V)
                def _():
                    rowid = i * _L + lax.iota(jnp.int32, _L)

                    def col(c, _):
                        cc = jnp.full((_L,), c, jnp.int32)
                        vals = plsc.load_gather(ne_v, [p, cc], mask=m)
                        plsc.store_scatter(rows_v, [rowid, cc], vals, mask=m)
                        return 0

                    lax.fori_loop(0, D, col, 0)

                return 0

            lax.fori_loop(0, _CH // _L, fixup, 0)

            pltpu.sync_copy(rows_v, out_hbm.at[pl.ds(cbase, _CH)])
            return 0

        lax.fori_loop(0, n_chunks, chunk_body, 0)

    return k


@jax.jit
def kernel(input, embed_weight, new_embed_weight):
    B_, S_ = input.shape
    V, D = embed_weight.shape
    NP = new_embed_weight.shape[0]
    B = B_ * S_
    flat_idx = input.reshape(B)
    k = _make_kernel(B, V, NP, D)
    out = k(flat_idx, embed_weight, new_embed_weight)
    return out.reshape(B_, S_, D)


# speedup vs baseline: 1.0766x; 1.0766x over previous
import functools

import jax
import jax.numpy as jnp
from jax import lax
from jax.experimental import pallas as pl
from jax.experimental.pallas import tpu as pltpu
from jax.experimental.pallas import tpu_sc as plsc

_NC = 2
_NS = 16
_NW = _NC * _NS
_L = 16

_CH = 512
_IDX_PER_DMA = 128


def _make_kernel(B, V, NP, D):
    per_w = B // _NW
    G = per_w // _CH
    n_dma = _CH // _IDX_PER_DMA
    assert per_w % _CH == 0 and G >= 4 and (G - 2) % 4 == 0
    mesh = plsc.VectorSubcoreMesh(core_axis_name="c", subcore_axis_name="s")

    @functools.partial(
        pl.kernel,
        out_type=jax.ShapeDtypeStruct((B, D), jnp.float32),
        mesh=mesh,
        compiler_params=pltpu.CompilerParams(
            needs_layout_passes=False, use_tc_tiling_on_sc=False
        ),
        scratch_types=[
            [pltpu.VMEM((_CH,), jnp.int32) for _ in range(4)],
            [pltpu.VMEM((_CH,), jnp.int32) for _ in range(2)],
            [pltpu.VMEM((_CH, D), jnp.float32) for _ in range(2)],
            pltpu.VMEM((NP, D), jnp.float32),
            [pltpu.SemaphoreType.DMA for _ in range(4)],
            [pltpu.SemaphoreType.DMA for _ in range(2)],
            [pltpu.SemaphoreType.DMA for _ in range(2)],
        ],
    )
    def k(idx_hbm, embed_hbm, ne_hbm, out_hbm, Q, S, R, ne_v, si, sg, so):
        wid = lax.axis_index("s") * _NC + lax.axis_index("c")
        base = wid * per_w
        pltpu.sync_copy(ne_hbm, ne_v)

        def fire_idx(g, q):
            pltpu.async_copy(idx_hbm.at[pl.ds(base + g * _CH, _CH)], Q[q], si[q])

        def wait_idx(q):
            pltpu.make_async_copy(idx_hbm.at[pl.ds(0, _CH)], Q[q], si[q]).wait()

        def remap(q, b):
            def step(i, _):
                v = Q[q][pl.ds(i * _L, _L)]
                S[b][pl.ds(i * _L, _L)] = jnp.where(v >= V, 0, v)
                return 0

            lax.fori_loop(0, _CH // _L, step, 0, unroll=4)

        def fire_gathers(g, b):
            for j in range(n_dma):
                pltpu.async_copy(
                    embed_hbm.at[S[b].at[pl.ds(j * _IDX_PER_DMA, _IDX_PER_DMA)]],
                    R[b].at[pl.ds(j * _IDX_PER_DMA, _IDX_PER_DMA)],
                    sg[b],
                )

        def wait_gathers(b):
            pltpu.make_async_copy(out_hbm.at[pl.ds(0, _CH)], R[b], sg[b]).wait()

        def fixup(q, b):
            def step(i, _):
                v = Q[q][pl.ds(i * _L, _L)]
                m = v >= V
                p = jnp.where(m, v - V, 0)

                @pl.when(jnp.max(v) >= V)
                def _():
                    rowid = i * _L + lax.iota(jnp.int32, _L)

                    def col(c, _):
                        cc = jnp.full((_L,), c, jnp.int32)
                        vals = plsc.load_gather(ne_v, [p, cc], mask=m)
                        plsc.store_scatter(R[b], [rowid, cc], vals, mask=m)
                        return 0

                    lax.fori_loop(0, D, col, 0)

                return 0

            lax.fori_loop(0, _CH // _L, step, 0)

        def fire_store(g, b):
            pltpu.async_copy(R[b], out_hbm.at[pl.ds(base + g * _CH, _CH)], so[b])

        def wait_store(b):
            pltpu.make_async_copy(R[b], out_hbm.at[pl.ds(0, _CH)], so[b]).wait()

        def body(g, q, b, prefetch):
            p, qp = 1 - b, (q - 1) % 4
            wait_idx(q)
            remap(q, b)
            wait_store(b)
            fire_gathers(g, b)
            wait_gathers(p)
            fixup(qp, p)
            fire_store(g - 1, p)
            if prefetch:
                fire_idx(g + 2, (q + 2) % 4)

        fire_idx(0, 0)
        fire_idx(1, 1)
        wait_idx(0)
        remap(0, 0)
        fire_gathers(0, 0)
        fire_idx(2, 2)
        wait_idx(1)
        remap(1, 1)
        fire_gathers(1, 1)
        wait_gathers(0)
        fixup(0, 0)
        fire_store(0, 0)
        fire_idx(3, 3)

        def quad(kk, _):
            for j in range(4):
                body(2 + kk * 4 + j, (2 + j) % 4, j % 2, True)
            return 0

        lax.fori_loop(0, (G - 2) // 4 - 1, quad, 0)

        for j in range(4):
            g = G - 4 + j
            body(g, (g % 4), g % 2, g + 2 <= G - 1)

        bl = (G - 1) % 2
        wait_gathers(bl)
        fixup((G - 1) % 4, bl)
        fire_store(G - 1, bl)
        wait_store(1 - bl)
        wait_store(bl)

    return k


@jax.jit
def kernel(input, embed_weight, new_embed_weight):
    B_, S_ = input.shape
    V, D = embed_weight.shape
    NP = new_embed_weight.shape[0]
    B = B_ * S_
    flat_idx = input.reshape(B)
    k = _make_kernel(B, V, NP, D)
    out = k(flat_idx, embed_weight, new_embed_weight)
    return out.reshape(B_, S_, D)
